# initial kernel scaffold (unmeasured)
import jax
import jax.numpy as jnp
from jax import lax
from jax.experimental import pallas as pl
from jax.experimental.pallas import tpu as pltpu

N_DEV = 16
B = 16
H = 16
D = 64
BS = 16
NPG = 128
NKEY = NPG * BS
NEG = -1e30


def kernel(Q, K, V, bt, lens):
    lens2 = lens.reshape(B, 1).astype(jnp.int32)

    def body(q_ref, k_ref, v_ref, bt_ref, lens_ref, out_ref,
             mine_ref, comm_ref, send_sems, recv_sems):
        my = lax.axis_index("i")

        barrier = pltpu.get_barrier_semaphore()
        for off in range(1, N_DEV):
            pl.semaphore_signal(
                barrier, inc=1,
                device_id=((my + off) % N_DEV,),
                device_id_type=pl.DeviceIdType.MESH,
            )
        pl.semaphore_wait(barrier, N_DEV - 1)

        bt_l = bt_ref[...]
        valid = lax.broadcasted_iota(jnp.int32, (B, NPG), 1) < lens_ref[...]
        page0 = my * NPG
        p_iota = lax.broadcasted_iota(jnp.int32, (1, 1, NPG), 2)
        eq = (bt_l[:, :, None] == page0 + p_iota) & valid[:, :, None]
        counts = jnp.sum(eq.astype(jnp.int32), axis=1)
        wk = jnp.broadcast_to(
            counts.astype(jnp.float32)[:, :, None], (B, NPG, BS)
        ).reshape(B, NKEY)

        q = q_ref[:, 0, :, :]
        k2 = k_ref[...].reshape(NKEY, H, D)
        v2 = v_ref[...].reshape(NKEY, H, D)
        s = lax.dot_general(
            q, k2, (((2,), (2,)), ((1,), (1,))),
            preferred_element_type=jnp.float32,
        ) * (D ** -0.5)
        mask = (wk > 0.0)[None, :, :]
        s = jnp.where(mask, s, NEG)
        m = jnp.max(s, axis=2)
        e = jnp.exp(s - m[:, :, None]) * wk[None, :, :]
        l = jnp.sum(e, axis=2)
        acc = lax.dot_general(
            e, v2, (((2,), (0,)), ((0,), (1,))),
            preferred_element_type=jnp.float32,
        )

        mine_ref[pl.ds(0, H), :, :] = acc
        ml = jnp.concatenate([m, l], axis=1)
        mine_ref[H, :, pl.ds(0, 2 * B)] = ml

        rdmas = []
        for off in range(1, N_DEV):
            slot = off - 1
            rdma = pltpu.make_async_remote_copy(
                src_ref=mine_ref,
                dst_ref=comm_ref.at[slot],
                send_sem=send_sems.at[slot],
                recv_sem=recv_sems.at[slot],
                device_id=((my + off) % N_DEV,),
                device_id_type=pl.DeviceIdType.MESH,
            )
            rdma.start()
            rdmas.append(rdma)

        for rdma in rdmas:
            rdma.wait_recv()

        allc = comm_ref[...]
        acc_r = allc[:, :H, :, :]
        m_r = allc[:, H, :, 0:B]
        l_r = allc[:, H, :, B:2 * B]

        gmax = jnp.maximum(m, jnp.max(m_r, axis=0))
        w0 = jnp.exp(m - gmax)
        wr = jnp.exp(m_r - gmax[None, :, :])
        num = acc * w0[:, :, None] + jnp.sum(
            acc_r * wr[:, :, :, None], axis=0
        )
        den = l * w0 + jnp.sum(l_r * wr, axis=0)
        o = num / den[:, :, None]
        out_ref[:, 0, :, :] = jnp.transpose(o, (1, 0, 2))

        for rdma in rdmas:
            rdma.wait_send()

    return pl.pallas_call(
        body,
        out_shape=jax.ShapeDtypeStruct((B, 1, H, D), jnp.float32),
        in_specs=[
            pl.BlockSpec(memory_space=pltpu.VMEM),
            pl.BlockSpec(memory_space=pltpu.VMEM),
            pl.BlockSpec(memory_space=pltpu.VMEM),
            pl.BlockSpec(memory_space=pltpu.VMEM),
            pl.BlockSpec(memory_space=pltpu.VMEM),
        ],
        out_specs=pl.BlockSpec(memory_space=pltpu.VMEM),
        scratch_shapes=[
            pltpu.VMEM((H + 1, B, D), jnp.float32),
            pltpu.VMEM((N_DEV - 1, H + 1, B, D), jnp.float32),
            pltpu.SemaphoreType.DMA((N_DEV - 1,)),
            pltpu.SemaphoreType.DMA((N_DEV - 1,)),
        ],
        compiler_params=pltpu.CompilerParams(collective_id=0),
    )(Q, K, V, bt, lens2)


# baseline (device time: 107501 ns/iter reference)
import jax
import jax.numpy as jnp
from jax import lax
from jax.experimental import pallas as pl
from jax.experimental.pallas import tpu as pltpu

N_DEV = 16
B = 16
H = 16
D = 64
BS = 16
NPG = 128
CP = 16
CKEY = CP * BS
NCH = NPG // CP
NEG = -1e30


def kernel(Q, K, V, bt, lens):
    lens2 = lens.reshape(B, 1).astype(jnp.int32)

    def body(q_ref, k_ref, v_ref, bt_ref, lens_ref, out_ref,
             mine_ref, comm_ref, send_sems, recv_sems):
        my = lax.axis_index("i")

        barrier = pltpu.get_barrier_semaphore()
        for off in range(1, N_DEV):
            pl.semaphore_signal(
                barrier, inc=1,
                device_id=((my + off) % N_DEV,),
                device_id_type=pl.DeviceIdType.MESH,
            )
        pl.semaphore_wait(barrier, N_DEV - 1)

        bt_l = bt_ref[...]
        valid = lax.broadcasted_iota(jnp.int32, (B, NPG), 1) < lens_ref[...]
        bt_v = jnp.where(valid, bt_l, -1)
        page0 = my * NPG
        p_iota = lax.broadcasted_iota(jnp.int32, (1, 1, NPG), 2)
        eq = bt_v[:, :, None] == (page0 + p_iota)
        counts = jnp.sum(eq.astype(jnp.float32), axis=1)

        q = q_ref[:, 0, :, :]
        m_run = jnp.full((H, B, 1), NEG, dtype=jnp.float32)
        l_run = jnp.zeros((H, B, 1), dtype=jnp.float32)
        acc_run = jnp.zeros((H, B, D), dtype=jnp.float32)
        for c in range(NCH):
            kc = k_ref[pl.ds(c * CP, CP)].reshape(CKEY, H, D)
            vc = v_ref[pl.ds(c * CP, CP)].reshape(CKEY, H, D)
            wk = jnp.broadcast_to(
                counts[:, c * CP:(c + 1) * CP, None], (B, CP, BS)
            ).reshape(B, CKEY)
            s = lax.dot_general(
                q, kc, (((2,), (2,)), ((1,), (1,))),
                preferred_element_type=jnp.float32,
            ) * (D ** -0.5)
            s = jnp.where((wk > 0.0)[None, :, :], s, NEG)
            m_new = jnp.maximum(m_run, jnp.max(s, axis=2, keepdims=True))
            alpha = jnp.exp(m_run - m_new)
            e = jnp.exp(s - m_new) * wk[None, :, :]
            l_run = l_run * alpha + jnp.sum(e, axis=2, keepdims=True)
            acc_run = acc_run * alpha + lax.dot_general(
                e, vc, (((2,), (0,)), ((0,), (1,))),
                preferred_element_type=jnp.float32,
            )
            m_run = m_new

        mine_ref[:, :, 0:D] = acc_run
        mine_ref[:, :, D:D + 1] = m_run
        mine_ref[:, :, D + 1:D + 2] = l_run

        rdmas = []
        for off in range(1, N_DEV):
            slot = off - 1
            rdma = pltpu.make_async_remote_copy(
                src_ref=mine_ref,
                dst_ref=comm_ref.at[slot],
                send_sem=send_sems.at[slot],
                recv_sem=recv_sems.at[slot],
                device_id=((my + off) % N_DEV,),
                device_id_type=pl.DeviceIdType.MESH,
            )
            rdma.start()
            rdmas.append(rdma)

        for rdma in rdmas:
            rdma.wait_recv()

        allc = comm_ref[...]
        acc_r = allc[:, :, :, 0:D]
        m_r = allc[:, :, :, D:D + 1]
        l_r = allc[:, :, :, D + 1:D + 2]

        gmax = jnp.maximum(m_run, jnp.max(m_r, axis=0))
        w0 = jnp.exp(m_run - gmax)
        wr = jnp.exp(m_r - gmax[None])
        num = acc_run * w0 + jnp.sum(acc_r * wr, axis=0)
        den = l_run * w0 + jnp.sum(l_r * wr, axis=0)
        o = num / den
        out_ref[:, 0, :, :] = jnp.transpose(o, (1, 0, 2))

        for rdma in rdmas:
            rdma.wait_send()

    return pl.pallas_call(
        body,
        out_shape=jax.ShapeDtypeStruct((B, 1, H, D), jnp.float32),
        in_specs=[
            pl.BlockSpec(memory_space=pltpu.VMEM),
            pl.BlockSpec(memory_space=pltpu.VMEM),
            pl.BlockSpec(memory_space=pltpu.VMEM),
            pl.BlockSpec(memory_space=pltpu.VMEM),
            pl.BlockSpec(memory_space=pltpu.VMEM),
        ],
        out_specs=pl.BlockSpec(memory_space=pltpu.VMEM),
        scratch_shapes=[
            pltpu.VMEM((H, B, D + 2), jnp.float32),
            pltpu.VMEM((N_DEV - 1, H, B, D + 2), jnp.float32),
            pltpu.SemaphoreType.DMA((N_DEV - 1,)),
            pltpu.SemaphoreType.DMA((N_DEV - 1,)),
        ],
        compiler_params=pltpu.CompilerParams(collective_id=0),
    )(Q, K, V, bt, lens2)


# device time: 60679 ns/iter; 1.7716x vs baseline; 1.7716x over previous
import jax
import jax.numpy as jnp
from jax import lax
from jax.experimental import pallas as pl
from jax.experimental.pallas import tpu as pltpu

N_DEV = 16
B = 16
H = 16
D = 64
BS = 16
NPG = 128
NKEY = NPG * BS
CP = 16
CKEY = CP * BS
NCH = NPG // CP
NEG = -1e30


def kernel(Q, K, V, bt, lens):
    lens2 = lens.reshape(B, 1).astype(jnp.int32)
    Kt = jnp.transpose(K.reshape(NKEY, H, D), (1, 0, 2))
    Vt = jnp.transpose(V.reshape(NKEY, H, D), (1, 0, 2))

    def body(q_ref, k_ref, v_ref, bt_ref, lens_ref, out_ref,
             mine_ref, comm_ref, send_sems, recv_sems):
        my = lax.axis_index("i")

        barrier = pltpu.get_barrier_semaphore()
        for off in range(1, N_DEV):
            pl.semaphore_signal(
                barrier, inc=1,
                device_id=((my + off) % N_DEV,),
                device_id_type=pl.DeviceIdType.MESH,
            )
        pl.semaphore_wait(barrier, N_DEV - 1)

        bt_l = bt_ref[...]
        valid = lax.broadcasted_iota(jnp.int32, (B, NPG), 1) < lens_ref[...]
        bt_v = jnp.where(valid, bt_l, -1)
        page0 = my * NPG
        p_iota = lax.broadcasted_iota(jnp.int32, (1, 1, CP), 2)

        qt = jnp.transpose(q_ref[:, 0, :, :], (1, 0, 2))

        def chunk(c, carry):
            m_run, l_run, acc_run = carry
            kc = k_ref[:, pl.ds(c * CKEY, CKEY), :]
            vc = v_ref[:, pl.ds(c * CKEY, CKEY), :]
            eq = bt_v[:, :, None] == (page0 + c * CP + p_iota)
            counts_c = jnp.sum(eq.astype(jnp.float32), axis=1)
            wk = jnp.broadcast_to(
                counts_c[:, :, None], (B, CP, BS)
            ).reshape(B, CKEY)
            s = lax.dot_general(
                qt, kc, (((2,), (2,)), ((0,), (0,))),
                preferred_element_type=jnp.float32,
            ) * (D ** -0.5)
            s = jnp.where((wk > 0.0)[None, :, :], s, NEG)
            m_new = jnp.maximum(m_run, jnp.max(s, axis=2, keepdims=True))
            alpha = jnp.exp(m_run - m_new)
            e = jnp.exp(s - m_new) * wk[None, :, :]
            l_new = l_run * alpha + jnp.sum(e, axis=2, keepdims=True)
            acc_new = acc_run * alpha + lax.dot_general(
                e, vc, (((2,), (1,)), ((0,), (0,))),
                preferred_element_type=jnp.float32,
            )
            return m_new, l_new, acc_new

        m_run, l_run, acc_run = lax.fori_loop(
            0, NCH, chunk,
            (jnp.full((H, B, 1), NEG, dtype=jnp.float32),
             jnp.zeros((H, B, 1), dtype=jnp.float32),
             jnp.zeros((H, B, D), dtype=jnp.float32)),
        )

        mine_ref[:, :, 0:D] = acc_run
        mine_ref[:, :, D:D + 1] = m_run
        mine_ref[:, :, D + 1:D + 2] = l_run

        rdmas = []
        for off in range(1, N_DEV):
            slot = off - 1
            rdma = pltpu.make_async_remote_copy(
                src_ref=mine_ref,
                dst_ref=comm_ref.at[slot],
                send_sem=send_sems.at[slot],
                recv_sem=recv_sems.at[slot],
                device_id=((my + off) % N_DEV,),
                device_id_type=pl.DeviceIdType.MESH,
            )
            rdma.start()
            rdmas.append(rdma)

        for rdma in rdmas:
            rdma.wait_recv()

        allc = comm_ref[...]
        acc_r = allc[:, :, :, 0:D]
        m_r = allc[:, :, :, D:D + 1]
        l_r = allc[:, :, :, D + 1:D + 2]

        gmax = jnp.maximum(m_run, jnp.max(m_r, axis=0))
        w0 = jnp.exp(m_run - gmax)
        wr = jnp.exp(m_r - gmax[None])
        num = acc_run * w0 + jnp.sum(acc_r * wr, axis=0)
        den = l_run * w0 + jnp.sum(l_r * wr, axis=0)
        o = num / den
        out_ref[:, 0, :, :] = jnp.transpose(o, (1, 0, 2))

        for rdma in rdmas:
            rdma.wait_send()

    return pl.pallas_call(
        body,
        out_shape=jax.ShapeDtypeStruct((B, 1, H, D), jnp.float32),
        in_specs=[
            pl.BlockSpec(memory_space=pltpu.VMEM),
            pl.BlockSpec(memory_space=pltpu.VMEM),
            pl.BlockSpec(memory_space=pltpu.VMEM),
            pl.BlockSpec(memory_space=pltpu.VMEM),
            pl.BlockSpec(memory_space=pltpu.VMEM),
        ],
        out_specs=pl.BlockSpec(memory_space=pltpu.VMEM),
        scratch_shapes=[
            pltpu.VMEM((H, B, D + 2), jnp.float32),
            pltpu.VMEM((N_DEV - 1, H, B, D + 2), jnp.float32),
            pltpu.SemaphoreType.DMA((N_DEV - 1,)),
            pltpu.SemaphoreType.DMA((N_DEV - 1,)),
        ],
        compiler_params=pltpu.CompilerParams(collective_id=0),
    )(Q, Kt, Vt, bt, lens2)


# device time: 44572 ns/iter; 2.4119x vs baseline; 1.3614x over previous
import jax
import jax.numpy as jnp
from jax import lax
from jax.experimental import pallas as pl
from jax.experimental.pallas import tpu as pltpu

N_DEV = 16
B = 16
H = 16
D = 64
BS = 16
NPG = 128
NKEY = NPG * BS
CP = 16
CKEY = CP * BS
NCH = NPG // CP
NEG = -1e30


def kernel(Q, K, V, bt, lens):
    lens2 = lens.reshape(B, 1).astype(jnp.int32)
    K4 = jnp.transpose(K, (1, 2, 3, 0))
    V4 = jnp.transpose(V, (1, 2, 3, 0))

    def body(q_ref, k_ref, v_ref, bt_ref, lens_ref, out_ref,
             mine_ref, comm_ref, send_sems, recv_sems):
        my = lax.axis_index("i")

        barrier = pltpu.get_barrier_semaphore()
        for off in range(1, N_DEV):
            pl.semaphore_signal(
                barrier, inc=1,
                device_id=((my + off) % N_DEV,),
                device_id_type=pl.DeviceIdType.MESH,
            )
        pl.semaphore_wait(barrier, N_DEV - 1)

        bt_l = bt_ref[...]
        valid = lax.broadcasted_iota(jnp.int32, (B, NPG), 1) < lens_ref[...]
        bt_v = jnp.where(valid, bt_l, -1)
        page0 = my * NPG
        p_iota = lax.broadcasted_iota(jnp.int32, (1, 1, NPG), 2)
        eq = bt_v[:, :, None] == (page0 + p_iota)
        counts = jnp.sum(eq.astype(jnp.float32), axis=1, keepdims=True)
        counts = jnp.transpose(counts, (1, 0, 2))
        maskc = counts > 0.0

        qt = jnp.transpose(q_ref[:, 0, :, :], (1, 0, 2))

        def tok(t, carry):
            m_run, l_run, acc_run = carry
            kc = k_ref[t]
            vc = v_ref[t]
            s = lax.dot_general(
                qt, kc, (((2,), (1,)), ((0,), (0,))),
                preferred_element_type=jnp.float32,
            ) * (D ** -0.5)
            s = jnp.where(maskc, s, NEG)
            m_new = jnp.maximum(m_run, jnp.max(s, axis=2, keepdims=True))
            alpha = jnp.exp(m_run - m_new)
            e = jnp.exp(s - m_new) * counts
            l_new = l_run * alpha + jnp.sum(e, axis=2, keepdims=True)
            acc_new = acc_run * alpha + lax.dot_general(
                e, vc, (((2,), (2,)), ((0,), (0,))),
                preferred_element_type=jnp.float32,
            )
            return m_new, l_new, acc_new

        m_run, l_run, acc_run = lax.fori_loop(
            0, BS, tok,
            (jnp.full((H, B, 1), NEG, dtype=jnp.float32),
             jnp.zeros((H, B, 1), dtype=jnp.float32),
             jnp.zeros((H, B, D), dtype=jnp.float32)),
        )

        mine_ref[:, :, 0:D] = acc_run
        mine_ref[:, :, D:D + 1] = m_run
        mine_ref[:, :, D + 1:D + 2] = l_run

        rdmas = []
        for off in range(1, N_DEV):
            slot = off - 1
            rdma = pltpu.make_async_remote_copy(
                src_ref=mine_ref,
                dst_ref=comm_ref.at[slot],
                send_sem=send_sems.at[slot],
                recv_sem=recv_sems.at[slot],
                device_id=((my + off) % N_DEV,),
                device_id_type=pl.DeviceIdType.MESH,
            )
            rdma.start()
            rdmas.append(rdma)

        for rdma in rdmas:
            rdma.wait_recv()

        allc = comm_ref[...]
        acc_r = allc[:, :, :, 0:D]
        m_r = allc[:, :, :, D:D + 1]
        l_r = allc[:, :, :, D + 1:D + 2]

        gmax = jnp.maximum(m_run, jnp.max(m_r, axis=0))
        w0 = jnp.exp(m_run - gmax)
        wr = jnp.exp(m_r - gmax[None])
        num = acc_run * w0 + jnp.sum(acc_r * wr, axis=0)
        den = l_run * w0 + jnp.sum(l_r * wr, axis=0)
        o = num / den
        out_ref[:, 0, :, :] = jnp.transpose(o, (1, 0, 2))

        for rdma in rdmas:
            rdma.wait_send()

    return pl.pallas_call(
        body,
        out_shape=jax.ShapeDtypeStruct((B, 1, H, D), jnp.float32),
        in_specs=[
            pl.BlockSpec(memory_space=pltpu.VMEM),
            pl.BlockSpec(memory_space=pltpu.VMEM),
            pl.BlockSpec(memory_space=pltpu.VMEM),
            pl.BlockSpec(memory_space=pltpu.VMEM),
            pl.BlockSpec(memory_space=pltpu.VMEM),
        ],
        out_specs=pl.BlockSpec(memory_space=pltpu.VMEM),
        scratch_shapes=[
            pltpu.VMEM((H, B, D + 2), jnp.float32),
            pltpu.VMEM((N_DEV - 1, H, B, D + 2), jnp.float32),
            pltpu.SemaphoreType.DMA((N_DEV - 1,)),
            pltpu.SemaphoreType.DMA((N_DEV - 1,)),
        ],
        compiler_params=pltpu.CompilerParams(collective_id=0),
    )(Q, K4, V4, bt, lens2)
